# baseline (speedup 1.0000x reference)
import functools

import jax
import jax.numpy as jnp
from jax import lax
from jax.experimental import pallas as pl
from jax.experimental.pallas import tpu as pltpu
from jax.experimental.pallas import tpu_sc as plsc

NC, NS, L = 2, 16, 16
NW = NC * NS
NB = 16384
NS_B = NB // 2
H = 50
D = 128
V = 32
EPW = NS_B // NW
EB = 4
NBUF = 2
NCHUNK = EPW // EB
NGROUP = NCHUNK // NBUF

_mesh = plsc.VectorSubcoreMesh(
    core_axis_name="c", subcore_axis_name="s", num_cores=NC, num_subcores=NS
)


@functools.partial(
    pl.kernel,
    mesh=_mesh,
    compiler_params=pltpu.CompilerParams(use_tc_tiling_on_sc=True),
    out_type=jax.ShapeDtypeStruct((NS_B, H, D), jnp.float32),
    scratch_types=[
        pltpu.VMEM((EPW, H), jnp.int32),
        pltpu.VMEM_SHARED((V, D), jnp.float32),
    ]
    + [pltpu.VMEM((EB, H, D), jnp.float32)] * NBUF
    + [pltpu.SemaphoreType.DMA] * NBUF,
)
def _gather_rows(idx_hbm, table_hbm, out_hbm, idx_v, table_sp, b0, b1, s0, s1):
    cid = lax.axis_index("c")
    sid = lax.axis_index("s")
    wid = sid * NC + cid
    bufs = (b0, b1)
    ssem = (s0, s1)

    pltpu.sync_copy(idx_hbm.at[pl.ds(wid * EPW, EPW)], idx_v)

    @pl.when(sid == 0)
    def _stage_table():
        pltpu.sync_copy(table_hbm, table_sp)

    plsc.subcore_barrier()

    def group(j, carry):
        for b in range(NBUF):
            k = j * NBUF + b
            e0 = wid * EPW + k * EB

            @pl.when(j >= 1)
            def _wait_store():
                pltpu.make_async_copy(
                    bufs[b], out_hbm.at[pl.ds(0, EB)], ssem[b]).wait()

            gathers = []
            for e in range(EB):
                idx50 = idx_v.at[k * EB + e]
                gathers.append(
                    pltpu.async_copy(
                        table_sp.at[idx50], bufs[b].at[e], ssem[b]))
            for g in gathers:
                g.wait()
            pltpu.async_copy(bufs[b], out_hbm.at[pl.ds(e0, EB)], ssem[b])
        return carry

    lax.fori_loop(0, NGROUP, group, 0)
    for b in range(NBUF):
        pltpu.make_async_copy(bufs[b], out_hbm.at[pl.ds(0, EB)], ssem[b]).wait()


def kernel(x, weight):
    xi = x.astype(jnp.int32)
    w = weight.astype(jnp.float32)
    o0 = _gather_rows(xi[:NS_B], w)
    o1 = _gather_rows(xi[NS_B:], w)
    return jnp.concatenate([o0, o1], axis=0)


# speedup vs baseline: 1.8185x; 1.8185x over previous
import functools

import jax
import jax.numpy as jnp
from jax import lax
from jax.experimental import pallas as pl
from jax.experimental.pallas import tpu as pltpu
from jax.experimental.pallas import tpu_sc as plsc

NC, NS, L = 2, 16, 16
NW = NC * NS
NB = 16384
H = 50
D = 128
V = 32
EPW = NB // NW
EB = 4
NBUF = 2
NCHUNK = EPW // EB
NGROUP = NCHUNK // NBUF

_mesh = plsc.VectorSubcoreMesh(
    core_axis_name="c", subcore_axis_name="s", num_cores=NC, num_subcores=NS
)


@functools.partial(
    pl.kernel,
    mesh=_mesh,
    compiler_params=pltpu.CompilerParams(use_tc_tiling_on_sc=True),
    out_type=jax.ShapeDtypeStruct((NB, H, D), jnp.float32),
    scratch_types=[
        pltpu.VMEM((EPW, H), jnp.int32),
        pltpu.VMEM_SHARED((V, D), jnp.float32),
    ]
    + [pltpu.VMEM((EB, H, D), jnp.float32)] * NBUF
    + [pltpu.SemaphoreType.DMA] * NBUF,
)
def _gather_rows(idx_hbm, table_hbm, out_hbm, idx_v, table_sp, b0, b1, s0, s1):
    cid = lax.axis_index("c")
    sid = lax.axis_index("s")
    wid = sid * NC + cid
    bufs = (b0, b1)
    ssem = (s0, s1)

    pltpu.sync_copy(idx_hbm.at[pl.ds(wid * EPW, EPW)], idx_v)

    @pl.when(sid == 0)
    def _stage_table():
        pltpu.sync_copy(table_hbm, table_sp)

    plsc.subcore_barrier()

    def group(j, carry):
        for b in range(NBUF):
            k = j * NBUF + b
            e0 = wid * EPW + k * EB

            @pl.when(j >= 1)
            def _wait_store():
                pltpu.make_async_copy(
                    bufs[b], out_hbm.at[pl.ds(0, EB)], ssem[b]).wait()

            gathers = []
            for e in range(EB):
                idx50 = idx_v.at[k * EB + e]
                gathers.append(
                    pltpu.async_copy(
                        table_sp.at[idx50], bufs[b].at[e], ssem[b]))
            for g in gathers:
                g.wait()
            pltpu.async_copy(bufs[b], out_hbm.at[pl.ds(e0, EB)], ssem[b])
        return carry

    lax.fori_loop(0, NGROUP, group, 0)
    for b in range(NBUF):
        pltpu.make_async_copy(bufs[b], out_hbm.at[pl.ds(0, EB)], ssem[b]).wait()


def kernel(x, weight):
    return _gather_rows(x.astype(jnp.int32), weight.astype(jnp.float32))
